# Initial kernel scaffold; baseline (speedup 1.0000x reference)
import jax
import jax.numpy as jnp
from jax.experimental import pallas as pl


def kernel(x, edge_index, W, B):
    raise NotImplementedError("write your pallas kernel here")


# speedup vs baseline: 4.0144x; 4.0144x over previous
import functools

import jax
import jax.numpy as jnp
from jax import lax
from jax.experimental import pallas as pl
from jax.experimental.pallas import tpu as pltpu
from jax.experimental.pallas import tpu_sc as plsc

N_NODES = 10000
N_EDGES = 320000
D = 128

NC = 2
NS = 16
NW = NC * NS
CH = 128
K = -(-N_EDGES // (NW * CH))
E_PAD = NW * K * CH
N_PAD = 10240
ROWS_PER_TILE = N_PAD // NS


def _sc_aggregate(x, src, dst, zeros):
    mesh = plsc.VectorSubcoreMesh(core_axis_name="c", subcore_axis_name="s")

    @functools.partial(
        pl.kernel,
        out_type=(
            jax.ShapeDtypeStruct((NC, N_PAD, D), jnp.float32),
            jax.ShapeDtypeStruct((NC, N_PAD), jnp.float32),
        ),
        mesh=mesh,
        scratch_types=[
            pltpu.VMEM((K, CH), jnp.int32),
            pltpu.VMEM((K, CH), jnp.int32),
            pltpu.VMEM((CH, D), jnp.float32),
            pltpu.VMEM((CH,), jnp.float32),
            pltpu.VMEM_SHARED((N_PAD, D), jnp.float32),
            pltpu.VMEM_SHARED((N_PAD,), jnp.float32),
            pltpu.SemaphoreType.DMA,
        ],
    )
    def agg_kernel(x_hbm, src_hbm, dst_hbm, z_hbm, agg_out, deg_out,
                   src_v, dst_v, rows_v, ones_v, agg_s, deg_s, sem):
        c = lax.axis_index("c")
        s = lax.axis_index("s")
        wid = c * NS + s
        row0 = s * ROWS_PER_TILE

        pltpu.sync_copy(z_hbm, agg_s.at[pl.ds(row0, ROWS_PER_TILE)])
        for t in range(ROWS_PER_TILE // D):
            pltpu.sync_copy(z_hbm.at[0], deg_s.at[pl.ds(row0 + t * D, D)])

        pltpu.sync_copy(src_hbm.at[wid], src_v)
        pltpu.sync_copy(dst_hbm.at[wid], dst_v)

        for i in range(CH // 16):
            ones_v[pl.ds(i * 16, 16)] = jnp.ones((16,), jnp.float32)

        plsc.subcore_barrier()

        def chunk(j, carry):
            pltpu.async_copy(x_hbm.at[src_v.at[j]], rows_v, sem).wait()
            pltpu.sync_copy(rows_v, agg_s.at[dst_v.at[j]], add=True)
            pltpu.sync_copy(ones_v, deg_s.at[dst_v.at[j]], add=True)
            return carry

        lax.fori_loop(0, K, chunk, 0)

        plsc.subcore_barrier()

        pltpu.sync_copy(agg_s.at[pl.ds(row0, ROWS_PER_TILE)],
                        agg_out.at[c, pl.ds(row0, ROWS_PER_TILE)])
        pltpu.sync_copy(deg_s.at[pl.ds(row0, ROWS_PER_TILE)],
                        deg_out.at[c, pl.ds(row0, ROWS_PER_TILE)])

    return agg_kernel(x, src, dst, zeros)


def _tc_update(agg2, deg2, x_pad, W, B):
    BR = 512
    grid = (N_PAD // BR,)

    def body(agg_ref, deg_ref, x_ref, w_ref, b_ref, o_ref):
        agg = agg_ref[0] + agg_ref[1]
        dg = deg_ref[0] + deg_ref[1]
        dg = jnp.where(dg == 0.0, 1.0, dg)
        aggn = agg / dg[:, None]
        mm1 = lax.dot_general(aggn, w_ref[...], (((1,), (1,)), ((), ())),
                              preferred_element_type=jnp.float32,
                              precision=lax.Precision.HIGHEST)
        mm2 = lax.dot_general(x_ref[...], b_ref[...], (((1,), (1,)), ((), ())),
                              preferred_element_type=jnp.float32,
                              precision=lax.Precision.HIGHEST)
        o_ref[...] = mm1 + mm2

    return pl.pallas_call(
        body,
        grid=grid,
        in_specs=[
            pl.BlockSpec((NC, BR, D), lambda i: (0, i, 0)),
            pl.BlockSpec((NC, BR), lambda i: (0, i)),
            pl.BlockSpec((BR, D), lambda i: (i, 0)),
            pl.BlockSpec((D, D), lambda i: (0, 0)),
            pl.BlockSpec((D, D), lambda i: (0, 0)),
        ],
        out_specs=pl.BlockSpec((BR, D), lambda i: (i, 0)),
        out_shape=jax.ShapeDtypeStruct((N_PAD, D), jnp.float32),
    )(agg2, deg2, x_pad, W, B)


def kernel(x, edge_index, W, B):
    src = edge_index[0].astype(jnp.int32)
    dst = edge_index[1].astype(jnp.int32)
    pad = E_PAD - N_EDGES
    src = jnp.concatenate([src, jnp.zeros((pad,), jnp.int32)])
    dst = jnp.concatenate([dst, jnp.full((pad,), N_NODES, jnp.int32)])
    src = src.reshape(NW, K, CH)
    dst = dst.reshape(NW, K, CH)
    zeros = jnp.zeros((ROWS_PER_TILE, D), jnp.float32)

    agg2, deg2 = _sc_aggregate(x, src, dst, zeros)

    x_pad = jnp.zeros((N_PAD, D), jnp.float32).at[:N_NODES].set(x)
    out = _tc_update(agg2, deg2, x_pad, W, B)
    return out[:N_NODES]


# speedup vs baseline: 4.5058x; 1.1224x over previous
import functools

import jax
import jax.numpy as jnp
from jax import lax
from jax.experimental import pallas as pl
from jax.experimental.pallas import tpu as pltpu
from jax.experimental.pallas import tpu_sc as plsc

N_NODES = 10000
N_EDGES = 320000
D = 128
DH = D // 2

NC = 2
NS = 16
CB = 128
NBUF = 4
G = 16
K = G * (-(-N_EDGES // (NS * CB * G)))
NG = K // G
E_PAD = NS * K * CB
N_PAD = 10240
ROWS_PER_TILE = N_PAD // NS


def _sc_aggregate(x2cat, src5, dst4, zeros, zeros1):
    mesh = plsc.VectorSubcoreMesh(core_axis_name="c", subcore_axis_name="s")

    @functools.partial(
        pl.kernel,
        out_type=(
            jax.ShapeDtypeStruct((NC, N_PAD, DH), jnp.float32),
            jax.ShapeDtypeStruct((NC, N_PAD), jnp.float32),
        ),
        mesh=mesh,
        compiler_params=pltpu.CompilerParams(use_tc_tiling_on_sc=False),
        scratch_types=[
            pltpu.VMEM((G, CB), jnp.int32),
            pltpu.VMEM((G, CB), jnp.int32),
            pltpu.VMEM((NBUF, CB, DH), jnp.float32),
            pltpu.VMEM((CB,), jnp.float32),
            pltpu.VMEM_SHARED((N_PAD, DH), jnp.float32),
            pltpu.VMEM_SHARED((N_PAD,), jnp.float32),
            pltpu.SemaphoreType.DMA((NBUF,)),
            pltpu.SemaphoreType.DMA((NBUF,)),
            pltpu.SemaphoreType.DMA((NBUF,)),
        ],
    )
    def agg_kernel(x_hbm, src_hbm, dst_hbm, z_hbm, z1_hbm, agg_out, deg_out,
                   src_v, dst_v, rows_v, ones_v, agg_s, deg_s,
                   gsem, ssem, dsem):
        c = lax.axis_index("c")
        s = lax.axis_index("s")
        row0 = s * ROWS_PER_TILE

        pltpu.sync_copy(z_hbm, agg_s.at[pl.ds(row0, ROWS_PER_TILE)])
        pltpu.sync_copy(z1_hbm, deg_s.at[pl.ds(row0, ROWS_PER_TILE)])

        for i in range(CB // 16):
            ones_v[pl.ds(i * 16, 16)] = jnp.ones((16,), jnp.float32)

        plsc.subcore_barrier()

        def group(g, carry):
            base = g * G
            pltpu.sync_copy(src_hbm.at[c, s, pl.ds(base, G)], src_v)
            pltpu.sync_copy(dst_hbm.at[s, pl.ds(base, G)], dst_v)
            for b in range(NBUF - 1):
                pltpu.async_copy(x_hbm.at[src_v.at[b]], rows_v.at[b],
                                 gsem.at[b])
            for i in range(G):
                b = i % NBUF
                j = base + i
                pltpu.make_async_copy(
                    x_hbm.at[src_v.at[i]], rows_v.at[b], gsem.at[b]).wait()
                pltpu.async_copy(rows_v.at[b], agg_s.at[dst_v.at[i]],
                                 ssem.at[b], add=True)
                deg_cond = (j < K // 2) == (c == 0)

                @pl.when(deg_cond)
                def _():
                    pltpu.async_copy(ones_v, deg_s.at[dst_v.at[i]],
                                     dsem.at[b], add=True)

                ni = i + NBUF - 1
                if ni < G:
                    nb = ni % NBUF
                    if i > 0:
                        pltpu.make_async_copy(
                            rows_v.at[nb], agg_s.at[dst_v.at[i - 1]],
                            ssem.at[nb]).wait()
                    pltpu.async_copy(x_hbm.at[src_v.at[ni]], rows_v.at[nb],
                                     gsem.at[nb])
            for i in range(G - NBUF, G):
                pltpu.make_async_copy(
                    rows_v.at[i % NBUF], agg_s.at[dst_v.at[i]],
                    ssem.at[i % NBUF]).wait()
            for i in range(G):
                jj = base + i
                deg_cond = (jj < K // 2) == (c == 0)

                @pl.when(deg_cond)
                def _():
                    pltpu.make_async_copy(
                        ones_v, deg_s.at[dst_v.at[i]], dsem.at[i % NBUF]).wait()
            return carry

        lax.fori_loop(0, NG, group, 0)

        plsc.subcore_barrier()

        pltpu.sync_copy(agg_s.at[pl.ds(row0, ROWS_PER_TILE)],
                        agg_out.at[c, pl.ds(row0, ROWS_PER_TILE)])
        pltpu.sync_copy(deg_s.at[pl.ds(row0, ROWS_PER_TILE)],
                        deg_out.at[c, pl.ds(row0, ROWS_PER_TILE)])

    return agg_kernel(x2cat, src5, dst4, zeros, zeros1)


def _tc_update(agg2, deg2, x_pad, W, B):
    BR = 512
    grid = (N_PAD // BR,)

    def body(agg_ref, deg_ref, x_ref, w_ref, b_ref, o_ref):
        agg = jnp.concatenate([agg_ref[0], agg_ref[1]], axis=1)
        dg = deg_ref[0] + deg_ref[1]
        dg = jnp.where(dg == 0.0, 1.0, dg)
        aggn = agg / dg[:, None]
        mm1 = lax.dot_general(aggn, w_ref[...], (((1,), (1,)), ((), ())),
                              preferred_element_type=jnp.float32,
                              precision=lax.Precision.HIGHEST)
        mm2 = lax.dot_general(x_ref[...], b_ref[...], (((1,), (1,)), ((), ())),
                              preferred_element_type=jnp.float32,
                              precision=lax.Precision.HIGHEST)
        o_ref[...] = mm1 + mm2

    return pl.pallas_call(
        body,
        grid=grid,
        in_specs=[
            pl.BlockSpec((NC, BR, DH), lambda i: (0, i, 0)),
            pl.BlockSpec((NC, BR), lambda i: (0, i)),
            pl.BlockSpec((BR, D), lambda i: (i, 0)),
            pl.BlockSpec((D, D), lambda i: (0, 0)),
            pl.BlockSpec((D, D), lambda i: (0, 0)),
        ],
        out_specs=pl.BlockSpec((BR, D), lambda i: (i, 0)),
        out_shape=jax.ShapeDtypeStruct((N_PAD, D), jnp.float32),
    )(agg2, deg2, x_pad, W, B)


def kernel(x, edge_index, W, B):
    src = edge_index[0].astype(jnp.int32)
    dst = edge_index[1].astype(jnp.int32)
    pad = E_PAD - N_EDGES
    src = jnp.concatenate([src, jnp.zeros((pad,), jnp.int32)])
    dst = jnp.concatenate([dst, jnp.full((pad,), N_NODES, jnp.int32)])
    src4 = src.reshape(NS, K, CB)
    dst4 = dst.reshape(NS, K, CB)
    src5 = jnp.stack([src4, src4 + N_PAD])

    x_pad = jnp.zeros((N_PAD, D), jnp.float32).at[:N_NODES].set(x)
    x2cat = x_pad.reshape(N_PAD, 2, DH).swapaxes(0, 1).reshape(2 * N_PAD, DH)
    zeros = jnp.zeros((ROWS_PER_TILE, DH), jnp.float32)
    zeros1 = jnp.zeros((ROWS_PER_TILE,), jnp.float32)

    agg2, deg2 = _sc_aggregate(x2cat, src5, dst4, zeros, zeros1)
    out = _tc_update(agg2, deg2, x_pad, W, B)
    return out[:N_NODES]
